# Initial kernel scaffold; baseline (speedup 1.0000x reference)
import jax
import jax.numpy as jnp
from jax.experimental import pallas as pl


def kernel(inputs, table):
    raise NotImplementedError("write your pallas kernel here")


# speedup vs baseline: 2.1826x; 2.1826x over previous
import functools

import jax
import jax.numpy as jnp
from jax import lax
from jax.experimental import pallas as pl
from jax.experimental.pallas import tpu as pltpu
from jax.experimental.pallas import tpu_sc as plsc

EPS = 1e-07

_NC = 2
_NS = 16
_NW = _NC * _NS
_L = 16
_W = 128


@functools.lru_cache(maxsize=None)
def _make_sc_kernel(B, S, D, V):
    P = S - 1
    BPW = B // _NW
    C = 8
    NCHUNK = BPW // C
    RPC = C * S
    GS = 80
    NG = RPC // GS
    NGRP = (P + _L - 1) // _L
    SUP = _W // D
    assert B % (_NW * C) == 0 and RPC % GS == 0 and D == 2 * _L

    mesh = plsc.VectorSubcoreMesh(core_axis_name="c", subcore_axis_name="s")

    @functools.partial(
        pl.kernel,
        mesh=mesh,
        out_type=jax.ShapeDtypeStruct((B * P,), jnp.float32),
        scratch_types=[
            pltpu.VMEM((RPC,), jnp.int32),
            pltpu.VMEM((RPC,), jnp.int32),
            pltpu.VMEM((RPC, _W), jnp.float32),
            pltpu.VMEM((C * P,), jnp.float32),
            pltpu.SemaphoreType.DMA,
        ],
        compiler_params=pltpu.CompilerParams(needs_layout_passes=False),
    )
    def sc_part(hi_hbm, lo_hbm, table_hbm, out_hbm, hi_v, lo_v, rows_v, x_v, sem):
        wid = lax.axis_index("s") * _NC + lax.axis_index("c")
        lane = lax.iota(jnp.int32, _L)

        def chunk_body(c, carry):
            b0 = wid * BPW + c * C
            pltpu.sync_copy(hi_hbm.at[pl.ds(b0 * S, RPC)], hi_v)
            pltpu.sync_copy(lo_hbm.at[pl.ds(b0 * S, RPC)], lo_v)
            copies = [
                pltpu.async_copy(
                    table_hbm.at[hi_v.at[pl.ds(i * GS, GS)]],
                    rows_v.at[pl.ds(i * GS, GS)],
                    sem,
                )
                for i in range(NG)
            ]
            for cp in copies:
                cp.wait()

            def bb_body(bb, carry2):
                ubase = bb * S
                ub16 = jnp.full((_L,), 0, jnp.int32) + ubase
                ucol = plsc.load_gather(lo_v, [ub16])
                u0 = plsc.load_gather(rows_v, [ub16, ucol + lane])
                u1 = plsc.load_gather(rows_v, [ub16, ucol + (lane + _L)])
                squ = jnp.sum(u0 * u0 + u1 * u1)
                one_m_squ = 1.0 - squ

                rowidx = []
                vcol = []
                for g in range(NGRP):
                    j = jnp.minimum(lane + (g * _L + 1), P)
                    rowidx.append(ub16 + j)
                    vcol.append(plsc.load_gather(lo_v, [rowidx[g]]))
                acc_s = [jnp.zeros((_L,), jnp.float32) for _ in range(NGRP)]
                acc_v = [jnp.zeros((_L,), jnp.float32) for _ in range(NGRP)]
                for d in range(D):
                    rot = (lane + d) & (D - 1)
                    uu = plsc.load_gather(rows_v, [ub16, ucol + rot])
                    for g in range(NGRP):
                        vv = plsc.load_gather(rows_v, [rowidx[g], vcol[g] + rot])
                        diff = uu - vv
                        acc_s[g] = acc_s[g] + diff * diff
                        acc_v[g] = acc_v[g] + vv * vv

                ob16 = jnp.full((_L,), 0, jnp.int32) + bb * P
                for g in range(NGRP):
                    q = (acc_s[g] + acc_s[g]) / (one_m_squ * (1.0 - acc_v[g]))
                    x = 1.0 + q + EPS
                    p = lane + g * _L
                    mask = (p < P) if (g + 1) * _L > P else None
                    plsc.store_scatter(x_v, [ob16 + p], x, mask=mask)
                return carry2

            lax.fori_loop(0, C, bb_body, 0)
            pltpu.sync_copy(x_v, out_hbm.at[pl.ds(b0 * P, C * P)])
            return carry

        lax.fori_loop(0, NCHUNK, chunk_body, 0)

    return sc_part


def _tc_finish_body(x_ref, o_ref):
    x = x_ref[...]
    z = jnp.sqrt(x * x - 1.0)
    o_ref[...] = jnp.log(x + z)


def kernel(inputs, table):
    B, S = inputs.shape
    V, D = table.shape
    P = S - 1
    sup = _W // D
    idx_flat = inputs.reshape(B * S).astype(jnp.int32)
    hi = idx_flat // sup
    lo = (idx_flat - hi * sup) * D
    table_w = table.reshape(V // sup, _W)
    sc_part = _make_sc_kernel(B, S, D, V)
    x_flat = sc_part(hi, lo, table_w)
    x = x_flat.reshape(B, P)

    nblk = 16
    out = pl.pallas_call(
        _tc_finish_body,
        out_shape=jax.ShapeDtypeStruct((B, P), jnp.float32),
        grid=(nblk,),
        in_specs=[pl.BlockSpec((B // nblk, P), lambda i: (i, 0))],
        out_specs=pl.BlockSpec((B // nblk, P), lambda i: (i, 0)),
    )(x)
    return out


# speedup vs baseline: 3.1633x; 1.4493x over previous
import functools

import jax
import jax.numpy as jnp
from jax import lax
from jax.experimental import pallas as pl
from jax.experimental.pallas import tpu as pltpu
from jax.experimental.pallas import tpu_sc as plsc

EPS = 1e-07

_NC = 2
_NS = 16
_NW = _NC * _NS
_L = 16


@functools.lru_cache(maxsize=None)
def _make_sc_kernel(B, S, D, V):
    P = S - 1
    BPW = B // _NW
    C = 16
    NCHUNK = BPW // C
    RPC = C * S
    GS = 80
    NG = RPC // GS
    NGRP = (P + _L - 1) // _L
    assert B % (_NW * C) == 0 and RPC % GS == 0 and D == 2 * _L
    assert NCHUNK % 2 == 0 and NCHUNK >= 4

    mesh = plsc.VectorSubcoreMesh(core_axis_name="c", subcore_axis_name="s")

    @functools.partial(
        pl.kernel,
        mesh=mesh,
        out_type=jax.ShapeDtypeStruct((B * P,), jnp.float32),
        scratch_types=[
            pltpu.VMEM((RPC,), jnp.int32),
            pltpu.VMEM((RPC,), jnp.int32),
            pltpu.VMEM((RPC, D), jnp.float32),
            pltpu.VMEM((RPC, D), jnp.float32),
            pltpu.VMEM((C * P,), jnp.float32),
            pltpu.VMEM((C * P,), jnp.float32),
            pltpu.SemaphoreType.DMA,
            pltpu.SemaphoreType.DMA,
            pltpu.SemaphoreType.DMA,
            pltpu.SemaphoreType.DMA,
            pltpu.SemaphoreType.DMA,
            pltpu.SemaphoreType.DMA,
        ],
        compiler_params=pltpu.CompilerParams(
            needs_layout_passes=False, use_tc_tiling_on_sc=False
        ),
    )
    def sc_part(idx_hbm, table_hbm, out_hbm,
                idx0, idx1, rows0, rows1, x0, x1,
                isem0, isem1, gsem0, gsem1, osem0, osem1):
        idx_v = (idx0, idx1)
        rows_v = (rows0, rows1)
        x_v = (x0, x1)
        isem = (isem0, isem1)
        gsem = (gsem0, gsem1)
        osem = (osem0, osem1)

        wid = lax.axis_index("s") * _NC + lax.axis_index("c")
        lane = lax.iota(jnp.int32, _L)
        base = wid * BPW

        def issue_idx(c, ph):
            b0 = base + c * C
            return pltpu.async_copy(
                idx_hbm.at[pl.ds(b0 * S, RPC)], idx_v[ph], isem[ph])

        def fire_gathers(ph):
            for i in range(NG):
                pltpu.async_copy(
                    table_hbm.at[idx_v[ph].at[pl.ds(i * GS, GS)]],
                    rows_v[ph].at[pl.ds(i * GS, GS)],
                    gsem[ph],
                )

        def drain_gathers(ph):
            for i in range(NG):
                pltpu.make_async_copy(
                    table_hbm.at[idx_v[ph].at[pl.ds(i * GS, GS)]],
                    rows_v[ph].at[pl.ds(i * GS, GS)],
                    gsem[ph],
                ).wait()

        def compute(ph):
            rows = rows_v[ph]
            xout = x_v[ph]

            def bb_body(bb, carry2):
                ubase = bb * S
                ub16 = jnp.full((_L,), 0, jnp.int32) + ubase
                u0 = plsc.load_gather(rows, [ub16, lane])
                u1 = plsc.load_gather(rows, [ub16, lane + _L])
                squ = jnp.sum(u0 * u0 + u1 * u1)
                one_m_squ = 1.0 - squ

                rowidx = []
                for g in range(NGRP):
                    j = jnp.minimum(lane + (g * _L + 1), P)
                    rowidx.append(ub16 + j)
                acc_s = [jnp.zeros((_L,), jnp.float32) for _ in range(NGRP)]
                acc_v = [jnp.zeros((_L,), jnp.float32) for _ in range(NGRP)]
                for d in range(D):
                    rot = (lane + d) & (D - 1)
                    uu = plsc.load_gather(rows, [ub16, rot])
                    for g in range(NGRP):
                        vv = plsc.load_gather(rows, [rowidx[g], rot])
                        diff = uu - vv
                        acc_s[g] = acc_s[g] + diff * diff
                        acc_v[g] = acc_v[g] + vv * vv

                ob16 = jnp.full((_L,), 0, jnp.int32) + bb * P
                for g in range(NGRP):
                    q = (acc_s[g] + acc_s[g]) / (one_m_squ * (1.0 - acc_v[g]))
                    x = 1.0 + q + EPS
                    p = lane + g * _L
                    mask = (p < P) if (g + 1) * _L > P else None
                    plsc.store_scatter(xout, [ob16 + p], x, mask=mask)
                return carry2

            lax.fori_loop(0, C, bb_body, 0)

        def issue_out(c, ph):
            b0 = base + c * C
            return pltpu.async_copy(
                x_v[ph], out_hbm.at[pl.ds(b0 * P, C * P)], osem[ph])

        def wait_out(c, ph):
            b0 = base + c * C
            pltpu.make_async_copy(
                x_v[ph], out_hbm.at[pl.ds(b0 * P, C * P)], osem[ph]).wait()

        issue_idx(0, 0)
        issue_idx(1, 1)
        pltpu.make_async_copy(
            idx_hbm.at[pl.ds(base * S, RPC)], idx_v[0], isem[0]).wait()
        fire_gathers(0)

        def pair_body(i, carry):
            for ph in (0, 1):
                n = 2 * i + ph
                drain_gathers(ph)

                @pl.when(n + 1 < NCHUNK)
                def _():
                    pltpu.make_async_copy(
                        idx_hbm.at[pl.ds((base + (n + 1) * C) * S, RPC)],
                        idx_v[1 - ph], isem[1 - ph]).wait()
                    fire_gathers(1 - ph)

                @pl.when(n + 2 < NCHUNK)
                def _():
                    issue_idx(n + 2, ph)

                @pl.when(n >= 2)
                def _():
                    wait_out(n - 2, ph)

                compute(ph)
                issue_out(n, ph)
            return carry

        lax.fori_loop(0, NCHUNK // 2, pair_body, 0)
        wait_out(NCHUNK - 2, 0)
        wait_out(NCHUNK - 1, 1)

    return sc_part


def _tc_finish_body(x_ref, o_ref):
    x = x_ref[...]
    z = jnp.sqrt(x * x - 1.0)
    o_ref[...] = jnp.log(x + z)


def kernel(inputs, table):
    B, S = inputs.shape
    V, D = table.shape
    P = S - 1
    idx_flat = inputs.reshape(B * S).astype(jnp.int32)
    sc_part = _make_sc_kernel(B, S, D, V)
    x_flat = sc_part(idx_flat, table)
    x = x_flat.reshape(B, P)

    nblk = 16
    out = pl.pallas_call(
        _tc_finish_body,
        out_shape=jax.ShapeDtypeStruct((B, P), jnp.float32),
        grid=(nblk,),
        in_specs=[pl.BlockSpec((B // nblk, P), lambda i: (i, 0))],
        out_specs=pl.BlockSpec((B // nblk, P), lambda i: (i, 0)),
    )(x)
    return out


# speedup vs baseline: 4.8318x; 1.5275x over previous
import functools

import jax
import jax.numpy as jnp
from jax import lax
from jax.experimental import pallas as pl
from jax.experimental.pallas import tpu as pltpu
from jax.experimental.pallas import tpu_sc as plsc

EPS = 1e-07

_NC = 2
_NS = 16
_NW = _NC * _NS
_L = 16


@functools.lru_cache(maxsize=None)
def _make_sc_kernel(B, S, D, V):
    P = S - 1
    BPW = B // _NW
    C = 16
    NCHUNK = BPW // C
    RPC = C * S
    GS = 80
    NG = RPC // GS
    NGRP = (P + _L - 1) // _L
    assert B % (_NW * C) == 0 and RPC % GS == 0 and D == 2 * _L
    assert NCHUNK % 2 == 0 and NCHUNK >= 4

    mesh = plsc.VectorSubcoreMesh(core_axis_name="c", subcore_axis_name="s")

    @functools.partial(
        pl.kernel,
        mesh=mesh,
        out_type=jax.ShapeDtypeStruct((B * P,), jnp.float32),
        scratch_types=[
            pltpu.VMEM((RPC,), jnp.int32),
            pltpu.VMEM((RPC,), jnp.int32),
            pltpu.VMEM((RPC, D), jnp.float32),
            pltpu.VMEM((RPC, D), jnp.float32),
            pltpu.VMEM((C * P,), jnp.float32),
            pltpu.VMEM((C * P,), jnp.float32),
            pltpu.SemaphoreType.DMA,
            pltpu.SemaphoreType.DMA,
            pltpu.SemaphoreType.DMA,
            pltpu.SemaphoreType.DMA,
            pltpu.SemaphoreType.DMA,
            pltpu.SemaphoreType.DMA,
        ],
        compiler_params=pltpu.CompilerParams(
            needs_layout_passes=False, use_tc_tiling_on_sc=False
        ),
    )
    def sc_part(idx_hbm, table_hbm, out_hbm,
                idx0, idx1, rows0, rows1, x0, x1,
                isem0, isem1, gsem0, gsem1, osem0, osem1):
        idx_v = (idx0, idx1)
        rows_v = (rows0, rows1)
        x_v = (x0, x1)
        isem = (isem0, isem1)
        gsem = (gsem0, gsem1)
        osem = (osem0, osem1)

        wid = lax.axis_index("s") * _NC + lax.axis_index("c")
        lane = lax.iota(jnp.int32, _L)
        base = wid * BPW

        def issue_idx(c, ph):
            b0 = base + c * C
            return pltpu.async_copy(
                idx_hbm.at[pl.ds(b0 * S, RPC)], idx_v[ph], isem[ph])

        def fire_gathers(ph):
            for i in range(NG):
                pltpu.async_copy(
                    table_hbm.at[idx_v[ph].at[pl.ds(i * GS, GS)]],
                    rows_v[ph].at[pl.ds(i * GS, GS)],
                    gsem[ph],
                )

        def drain_gathers(ph):
            for i in range(NG):
                pltpu.make_async_copy(
                    table_hbm.at[idx_v[ph].at[pl.ds(i * GS, GS)]],
                    rows_v[ph].at[pl.ds(i * GS, GS)],
                    gsem[ph],
                ).wait()

        def compute(ph):
            rows = rows_v[ph]
            xout = x_v[ph]

            def bb_body(bb, carry2):
                ubase = bb * S
                ub16 = jnp.full((_L,), 0, jnp.int32) + ubase
                u0 = plsc.load_gather(rows, [ub16, lane])
                u1 = plsc.load_gather(rows, [ub16, lane + _L])
                squ = jnp.sum(u0 * u0 + u1 * u1)
                one_m_squ = 1.0 - squ

                rowidx = []
                for g in range(NGRP):
                    j = jnp.minimum(lane + (g * _L + 1), P)
                    rowidx.append(ub16 + j)
                acc_s = [jnp.zeros((_L,), jnp.float32) for _ in range(NGRP)]
                acc_v = [jnp.zeros((_L,), jnp.float32) for _ in range(NGRP)]
                for d in range(D):
                    rot = (lane + d) & (D - 1)
                    uu = plsc.load_gather(rows, [ub16, rot])
                    for g in range(NGRP):
                        vv = plsc.load_gather(rows, [rowidx[g], rot])
                        diff = uu - vv
                        acc_s[g] = acc_s[g] + diff * diff
                        acc_v[g] = acc_v[g] + vv * vv

                ob16 = jnp.full((_L,), 0, jnp.int32) + bb * P
                for g in range(NGRP):
                    q = (acc_s[g] + acc_s[g]) / (one_m_squ * (1.0 - acc_v[g]))
                    x = 1.0 + q + EPS
                    p = lane + g * _L
                    mask = (p < P) if (g + 1) * _L > P else None
                    plsc.store_scatter(xout, [ob16 + p], x, mask=mask)
                return carry2

            lax.fori_loop(0, C, bb_body, 0)

        def issue_out(c, ph):
            b0 = base + c * C
            return pltpu.async_copy(
                x_v[ph], out_hbm.at[pl.ds(b0 * P, C * P)], osem[ph])

        def wait_out(c, ph):
            b0 = base + c * C
            pltpu.make_async_copy(
                x_v[ph], out_hbm.at[pl.ds(b0 * P, C * P)], osem[ph]).wait()

        issue_idx(0, 0)
        issue_idx(1, 1)
        pltpu.make_async_copy(
            idx_hbm.at[pl.ds(base * S, RPC)], idx_v[0], isem[0]).wait()
        fire_gathers(0)

        def pair_body(i, carry):
            for ph in (0, 1):
                n = 2 * i + ph
                drain_gathers(ph)

                @pl.when(n + 1 < NCHUNK)
                def _():
                    pltpu.make_async_copy(
                        idx_hbm.at[pl.ds((base + (n + 1) * C) * S, RPC)],
                        idx_v[1 - ph], isem[1 - ph]).wait()
                    fire_gathers(1 - ph)

                @pl.when(n + 2 < NCHUNK)
                def _():
                    issue_idx(n + 2, ph)

                @pl.when(n >= 2)
                def _():
                    wait_out(n - 2, ph)

                compute(ph)
                issue_out(n, ph)
            return carry

        lax.fori_loop(0, NCHUNK // 2, pair_body, 0)
        wait_out(NCHUNK - 2, 0)
        wait_out(NCHUNK - 1, 1)

    return sc_part


def _tc_finish_body(x_ref, o_ref):
    x = x_ref[...]
    z = jnp.sqrt(x * x - 1.0)
    o_ref[...] = jnp.log(x + z)


def _tc_transpose_body(xt_ref, o_ref):
    x = xt_ref[...]
    d, w = x.shape
    br = w // 4
    o_ref[...] = jnp.concatenate(
        [jnp.transpose(x[:, c * br:(c + 1) * br]) for c in range(4)], axis=1)


def _linearize_table(table):
    V, D = table.shape
    BR = 2048
    W = 4 * BR
    assert 4 * D == 128
    table_t = jnp.swapaxes(table, 0, 1)
    nblk = (V + W - 1) // W
    lin = pl.pallas_call(
        _tc_transpose_body,
        out_shape=jax.ShapeDtypeStruct((nblk * BR, 4 * D), jnp.float32),
        grid=(nblk,),
        in_specs=[pl.BlockSpec((D, W), lambda i: (0, i))],
        out_specs=pl.BlockSpec((BR, 4 * D), lambda i: (i, 0)),
    )(table_t)
    return lin.reshape(nblk * W, D)


def kernel(inputs, table):
    B, S = inputs.shape
    V, D = table.shape
    P = S - 1
    idx_flat = inputs.reshape(B * S).astype(jnp.int32)
    lin = _linearize_table(table)
    br, w = 2048, 8192
    i = idx_flat // w
    rem = idx_flat - i * w
    c = rem // br
    t = rem - c * br
    idx_perm = (i * br + t) * 4 + c
    sc_part = _make_sc_kernel(B, S, D, lin.shape[0])
    x_flat = sc_part(idx_perm, lin)
    x = x_flat.reshape(B, P)

    nblk = 16
    out = pl.pallas_call(
        _tc_finish_body,
        out_shape=jax.ShapeDtypeStruct((B, P), jnp.float32),
        grid=(nblk,),
        in_specs=[pl.BlockSpec((B // nblk, P), lambda i: (i, 0))],
        out_specs=pl.BlockSpec((B // nblk, P), lambda i: (i, 0)),
    )(x)
    return out


# speedup vs baseline: 6.5312x; 1.3517x over previous
import functools

import jax
import jax.numpy as jnp
from jax import lax
from jax.experimental import pallas as pl
from jax.experimental.pallas import tpu as pltpu
from jax.experimental.pallas import tpu_sc as plsc

EPS = 1e-07

_NC = 2
_NS = 16
_NW = _NC * _NS
_L = 16


@functools.lru_cache(maxsize=None)
def _make_sc_kernel(B, S, D, V):
    P = S - 1
    BPW = B // _NW
    C = 16
    NCHUNK = BPW // C
    RPC = C * S
    GS = 80
    NG = RPC // GS
    NGRP = (P + _L - 1) // _L
    assert B % (_NW * C) == 0 and RPC % GS == 0 and D == 2 * _L
    assert NCHUNK % 2 == 0 and NCHUNK >= 4

    mesh = plsc.VectorSubcoreMesh(core_axis_name="c", subcore_axis_name="s")

    @functools.partial(
        pl.kernel,
        mesh=mesh,
        out_type=jax.ShapeDtypeStruct((B * P,), jnp.float32),
        scratch_types=[
            pltpu.VMEM((RPC,), jnp.int32),
            pltpu.VMEM((RPC,), jnp.int32),
            pltpu.VMEM((RPC, D), jnp.float32),
            pltpu.VMEM((RPC, D), jnp.float32),
            pltpu.VMEM((C * P,), jnp.float32),
            pltpu.VMEM((C * P,), jnp.float32),
            pltpu.SemaphoreType.DMA,
            pltpu.SemaphoreType.DMA,
            pltpu.SemaphoreType.DMA,
            pltpu.SemaphoreType.DMA,
            pltpu.SemaphoreType.DMA,
            pltpu.SemaphoreType.DMA,
        ],
        compiler_params=pltpu.CompilerParams(
            needs_layout_passes=False, use_tc_tiling_on_sc=False
        ),
    )
    def sc_part(idx_hbm, table_hbm, out_hbm,
                idx0, idx1, rows0, rows1, x0, x1,
                isem0, isem1, gsem0, gsem1, osem0, osem1):
        idx_v = (idx0, idx1)
        rows_v = (rows0, rows1)
        x_v = (x0, x1)
        isem = (isem0, isem1)
        gsem = (gsem0, gsem1)
        osem = (osem0, osem1)

        wid = lax.axis_index("s") * _NC + lax.axis_index("c")
        lane = lax.iota(jnp.int32, _L)
        base = wid * BPW

        def issue_idx(c, ph):
            b0 = base + c * C
            return pltpu.async_copy(
                idx_hbm.at[pl.ds(b0 * S, RPC)], idx_v[ph], isem[ph])

        def fire_gathers(ph):
            for i in range(NG):
                pltpu.async_copy(
                    table_hbm.at[idx_v[ph].at[pl.ds(i * GS, GS)]],
                    rows_v[ph].at[pl.ds(i * GS, GS)],
                    gsem[ph],
                )

        def drain_gathers(ph):
            for i in range(NG):
                pltpu.make_async_copy(
                    table_hbm.at[idx_v[ph].at[pl.ds(i * GS, GS)]],
                    rows_v[ph].at[pl.ds(i * GS, GS)],
                    gsem[ph],
                ).wait()

        def compute(ph):
            rows = rows_v[ph]
            xout = x_v[ph]

            def bb_body(bb, carry2):
                ubase = bb * S
                ub16 = jnp.full((_L,), 0, jnp.int32) + ubase
                u0 = plsc.load_gather(rows, [ub16, lane])
                u1 = plsc.load_gather(rows, [ub16, lane + _L])
                squ = jnp.sum(u0 * u0 + u1 * u1)
                one_m_squ = 1.0 - squ

                rowidx = []
                for g in range(NGRP):
                    j = jnp.minimum(lane + (g * _L + 1), P)
                    rowidx.append(ub16 + j)
                acc_s = [jnp.zeros((_L,), jnp.float32) for _ in range(NGRP)]
                acc_v = [jnp.zeros((_L,), jnp.float32) for _ in range(NGRP)]
                for d in range(D):
                    rot = (lane + d) & (D - 1)
                    uu = plsc.load_gather(rows, [ub16, rot])
                    for g in range(NGRP):
                        vv = plsc.load_gather(rows, [rowidx[g], rot])
                        diff = uu - vv
                        acc_s[g] = acc_s[g] + diff * diff
                        acc_v[g] = acc_v[g] + vv * vv

                ob16 = jnp.full((_L,), 0, jnp.int32) + bb * P
                for g in range(NGRP):
                    q = (acc_s[g] + acc_s[g]) / (one_m_squ * (1.0 - acc_v[g]))
                    x = 1.0 + q + EPS
                    p = lane + g * _L
                    mask = (p < P) if (g + 1) * _L > P else None
                    plsc.store_scatter(xout, [ob16 + p], x, mask=mask)
                return carry2

            lax.fori_loop(0, C, bb_body, 0)

        def issue_out(c, ph):
            b0 = base + c * C
            return pltpu.async_copy(
                x_v[ph], out_hbm.at[pl.ds(b0 * P, C * P)], osem[ph])

        def wait_out(c, ph):
            b0 = base + c * C
            pltpu.make_async_copy(
                x_v[ph], out_hbm.at[pl.ds(b0 * P, C * P)], osem[ph]).wait()

        issue_idx(0, 0)
        issue_idx(1, 1)
        pltpu.make_async_copy(
            idx_hbm.at[pl.ds(base * S, RPC)], idx_v[0], isem[0]).wait()
        fire_gathers(0)

        def pair_body(i, carry):
            for ph in (0, 1):
                n = 2 * i + ph
                drain_gathers(ph)

                @pl.when(n + 1 < NCHUNK)
                def _():
                    pltpu.make_async_copy(
                        idx_hbm.at[pl.ds((base + (n + 1) * C) * S, RPC)],
                        idx_v[1 - ph], isem[1 - ph]).wait()
                    fire_gathers(1 - ph)

                @pl.when(n + 2 < NCHUNK)
                def _():
                    issue_idx(n + 2, ph)

                @pl.when(n >= 2)
                def _():
                    wait_out(n - 2, ph)

                compute(ph)
                issue_out(n, ph)
            return carry

        lax.fori_loop(0, NCHUNK // 2, pair_body, 0)
        wait_out(NCHUNK - 2, 0)
        wait_out(NCHUNK - 1, 1)

    return sc_part


def _tc_finish_body(x_ref, o_ref):
    x = x_ref[...]
    z = jnp.sqrt(x * x - 1.0)
    o_ref[...] = jnp.log(x + z)


def _tc_transpose_body(xt_ref, o_ref):
    x = xt_ref[...]
    d, w = x.shape
    br = w // 4
    stack = jnp.concatenate(
        [x[:, c * br:(c + 1) * br] for c in range(4)], axis=0)
    n = 4 * d
    eye = (lax.broadcasted_iota(jnp.int32, (n, n), 0)
           == lax.broadcasted_iota(jnp.int32, (n, n), 1)).astype(jnp.float32)
    o_ref[...] = lax.dot_general(
        stack, eye, (((0,), (0,)), ((), ())),
        preferred_element_type=jnp.float32)


def _linearize_table(table):
    V, D = table.shape
    BR = 2048
    W = 4 * BR
    assert 4 * D == 128
    table_t = jnp.swapaxes(table, 0, 1)
    nblk = (V + W - 1) // W
    lin = pl.pallas_call(
        _tc_transpose_body,
        out_shape=jax.ShapeDtypeStruct((nblk * BR, 4 * D), jnp.float32),
        grid=(nblk,),
        in_specs=[pl.BlockSpec((D, W), lambda i: (0, i))],
        out_specs=pl.BlockSpec((BR, 4 * D), lambda i: (i, 0)),
    )(table_t)
    return lin.reshape(nblk * W, D)


def kernel(inputs, table):
    B, S = inputs.shape
    V, D = table.shape
    P = S - 1
    idx_flat = inputs.reshape(B * S).astype(jnp.int32)
    lin = _linearize_table(table)
    br, w = 2048, 8192
    i = idx_flat // w
    rem = idx_flat - i * w
    c = rem // br
    t = rem - c * br
    idx_perm = (i * br + t) * 4 + c
    sc_part = _make_sc_kernel(B, S, D, lin.shape[0])
    x_flat = sc_part(idx_perm, lin)
    x = x_flat.reshape(B, P)

    nblk = 16
    out = pl.pallas_call(
        _tc_finish_body,
        out_shape=jax.ShapeDtypeStruct((B, P), jnp.float32),
        grid=(nblk,),
        in_specs=[pl.BlockSpec((B // nblk, P), lambda i: (i, 0))],
        out_specs=pl.BlockSpec((B // nblk, P), lambda i: (i, 0)),
    )(x)
    return out
